# baseline (speedup 1.0000x reference)
import functools

import jax
import jax.numpy as jnp
from jax import lax
from jax.experimental import pallas as pl
from jax.experimental.pallas import tpu as pltpu
from jax.experimental.pallas import tpu_sc as plsc

NUM_CORES = 2
NUM_SUBCORES = 16
LANES = 16
NW = NUM_CORES * NUM_SUBCORES

B = 16384
D = 32
BPW = B // NW
CHUNK = 128
NCHUNK = BPW // CHUNK
NGROUP = BPW // LANES


def _body(ut_hbm, it_hbm, uidx_hbm, iidx_hbm, ubias_hbm, ibias_hbm,
          gbias_hbm, out_hbm,
          idx_u, idx_i, udv, idv, ubv, ibv, gbv, outv, sem):
    wid = lax.axis_index("s") * NUM_CORES + lax.axis_index("c")
    base = wid * BPW

    pltpu.sync_copy(uidx_hbm.at[pl.ds(base, BPW)], idx_u)
    pltpu.sync_copy(iidx_hbm.at[pl.ds(base, BPW)], idx_i)
    pltpu.sync_copy(gbias_hbm, gbv.at[pl.ds(0, 1)])

    pltpu.async_copy(ubias_hbm.at[idx_u], ubv, sem)
    pltpu.async_copy(ibias_hbm.at[idx_i], ibv, sem)

    def fire(d, _):
        pltpu.async_copy(ut_hbm.at[d].at[idx_u], udv.at[d], sem)
        pltpu.async_copy(it_hbm.at[d].at[idx_i], idv.at[d], sem)
        return 0

    lax.fori_loop(0, D, fire, 0)

    def drain(d, _):
        pltpu.make_async_copy(ut_hbm.at[d].at[idx_u], udv.at[d], sem).wait()
        pltpu.make_async_copy(it_hbm.at[d].at[idx_i], idv.at[d], sem).wait()
        return 0

    lax.fori_loop(0, D, drain, 0)
    pltpu.make_async_copy(ubias_hbm.at[idx_u], ubv, sem).wait()
    pltpu.make_async_copy(ibias_hbm.at[idx_i], ibv, sem).wait()

    gb = gbv[...][0]

    def group(g, _):
        sl = pl.ds(g * LANES, LANES)
        acc = ubv[sl] + ibv[sl] + gb
        for d in range(D):
            acc = acc + udv[d, sl] * idv[d, sl]
        outv[sl] = acc
        return 0

    lax.fori_loop(0, NGROUP, group, 0)
    pltpu.sync_copy(outv, out_hbm.at[pl.ds(base, BPW)])


@jax.jit
def _run(ut, it, user_idx, item_idx, user_bias, item_bias, global_bias):
    mesh = plsc.VectorSubcoreMesh(
        core_axis_name="c", subcore_axis_name="s",
        num_cores=NUM_CORES, num_subcores=NUM_SUBCORES)
    f = functools.partial(
        pl.kernel,
        out_type=jax.ShapeDtypeStruct((B,), jnp.float32),
        mesh=mesh,
        compiler_params=pltpu.CompilerParams(
            needs_layout_passes=False, use_tc_tiling_on_sc=False),
        scratch_types=[
            pltpu.VMEM((BPW,), jnp.int32),
            pltpu.VMEM((BPW,), jnp.int32),
            pltpu.VMEM((D, BPW), jnp.float32),
            pltpu.VMEM((D, BPW), jnp.float32),
            pltpu.VMEM((BPW,), jnp.float32),
            pltpu.VMEM((BPW,), jnp.float32),
            pltpu.VMEM((LANES,), jnp.float32),
            pltpu.VMEM((BPW,), jnp.float32),
            pltpu.SemaphoreType.DMA,
        ],
    )(_body)
    return f(ut, it, user_idx, item_idx, user_bias, item_bias, global_bias)


def kernel(user_idx, item_idx, user_embeddings, item_embeddings,
           user_bias, item_bias, global_bias):
    return _run(user_embeddings.T, item_embeddings.T,
                user_idx.astype(jnp.int32), item_idx.astype(jnp.int32),
                user_bias.reshape(-1), item_bias.reshape(-1), global_bias)


# speedup vs baseline: 2.0246x; 2.0246x over previous
import functools

import jax
import jax.numpy as jnp
from jax import lax
from jax.experimental import pallas as pl
from jax.experimental.pallas import tpu as pltpu
from jax.experimental.pallas import tpu_sc as plsc

NUM_CORES = 2
NUM_SUBCORES = 16
LANES = 16
NW = NUM_CORES * NUM_SUBCORES

B = 16384
D = 32
BPW = B // NW
CHUNK = 128
NCHUNK = BPW // CHUNK
NGROUP = BPW // LANES


def _body(ue_hbm, ie_hbm, uidx_hbm, iidx_hbm, ubias_hbm, ibias_hbm,
          gbias_hbm, out_hbm,
          idx_u, idx_i, urows, irows, ubv, ibv, gbv, part, outv, sem):
    wid = lax.axis_index("s") * NUM_CORES + lax.axis_index("c")
    base = wid * BPW

    pltpu.sync_copy(uidx_hbm.at[pl.ds(base, BPW)], idx_u)
    pltpu.sync_copy(iidx_hbm.at[pl.ds(base, BPW)], idx_i)
    pltpu.sync_copy(gbias_hbm, gbv.at[pl.ds(0, 1)])

    for c in range(NCHUNK):
        s = pl.ds(c * CHUNK, CHUNK)
        pltpu.async_copy(ue_hbm.at[idx_u.at[s]], urows.at[s], sem).wait()
        pltpu.async_copy(ie_hbm.at[idx_i.at[s]], irows.at[s], sem).wait()
        pltpu.async_copy(ubias_hbm.at[idx_u.at[s]], ubv.at[s], sem).wait()
        pltpu.async_copy(ibias_hbm.at[idx_i.at[s]], ibv.at[s], sem).wait()

    iota = lax.iota(jnp.int32, LANES)
    zeros = jnp.zeros((LANES,), jnp.int32)
    gb = plsc.load_gather(gbv, [zeros])

    def fold(p, _):
        u0 = urows[p, pl.ds(0, LANES)]
        u1 = urows[p, pl.ds(LANES, LANES)]
        i0 = irows[p, pl.ds(0, LANES)]
        i1 = irows[p, pl.ds(LANES, LANES)]
        part[p, pl.ds(0, LANES)] = u0 * i0 + u1 * i1
        return 0

    lax.fori_loop(0, BPW, fold, 0)

    def group(g, _):
        rows = g * LANES + iota
        acc = (plsc.load_gather(ubv, [rows, zeros]) +
               plsc.load_gather(ibv, [rows, zeros]) + gb)
        for k in range(LANES):
            cols = jnp.bitwise_and(iota + k, LANES - 1)
            acc = acc + plsc.load_gather(part, [rows, cols])
        outv[pl.ds(g * LANES, LANES)] = acc
        return 0

    lax.fori_loop(0, NGROUP, group, 0)
    pltpu.sync_copy(outv, out_hbm.at[pl.ds(base, BPW)])


@jax.jit
def _run(ue, ie, user_idx, item_idx, user_bias, item_bias, global_bias):
    mesh = plsc.VectorSubcoreMesh(
        core_axis_name="c", subcore_axis_name="s",
        num_cores=NUM_CORES, num_subcores=NUM_SUBCORES)
    f = functools.partial(
        pl.kernel,
        out_type=jax.ShapeDtypeStruct((B,), jnp.float32),
        mesh=mesh,
        compiler_params=pltpu.CompilerParams(
            needs_layout_passes=False, use_tc_tiling_on_sc=False),
        scratch_types=[
            pltpu.VMEM((BPW,), jnp.int32),
            pltpu.VMEM((BPW,), jnp.int32),
            pltpu.VMEM((BPW, D), jnp.float32),
            pltpu.VMEM((BPW, D), jnp.float32),
            pltpu.VMEM((BPW, 1), jnp.float32),
            pltpu.VMEM((BPW, 1), jnp.float32),
            pltpu.VMEM((LANES,), jnp.float32),
            pltpu.VMEM((BPW, LANES), jnp.float32),
            pltpu.VMEM((BPW,), jnp.float32),
            pltpu.SemaphoreType.DMA,
        ],
    )(_body)
    return f(ue, ie, user_idx, item_idx, user_bias, item_bias, global_bias)


def kernel(user_idx, item_idx, user_embeddings, item_embeddings,
           user_bias, item_bias, global_bias):
    return _run(user_embeddings, item_embeddings,
                user_idx.astype(jnp.int32), item_idx.astype(jnp.int32),
                user_bias, item_bias, global_bias)
